# baseline (speedup 1.0000x reference)
import functools
import jax
import jax.numpy as jnp
from jax.experimental import pallas as pl
from jax.experimental.pallas import tpu as pltpu

_SHAPE = (32, 2048, 64)


@functools.lru_cache(maxsize=1)
def _gumbel_noise():
    key = jax.random.key(42)
    u = jax.random.uniform(key, _SHAPE, dtype=jnp.float32)
    g = -jnp.log(-jnp.log(u + 1e-20) + 1e-20)
    return jax.block_until_ready(g)


def _hard_onehot_kernel(z_ref, o_ref):
    z = z_ref[...]
    m = jnp.max(z, axis=-1, keepdims=True)
    iota = jax.lax.broadcasted_iota(jnp.int32, z.shape, z.ndim - 1)
    idx = jnp.min(jnp.where(z == m, iota, z.shape[-1]), axis=-1, keepdims=True)
    o_ref[...] = (iota == idx).astype(jnp.float32)


def kernel(logits):
    B, N, K = logits.shape
    z = logits + _gumbel_noise()
    BI = 8
    out = pl.pallas_call(
        _hard_onehot_kernel,
        out_shape=jax.ShapeDtypeStruct((B, N, K), jnp.float32),
        grid=(B // BI,),
        in_specs=[pl.BlockSpec((BI, N, K), lambda i: (i, 0, 0))],
        out_specs=pl.BlockSpec((BI, N, K), lambda i: (i, 0, 0)),
        compiler_params=pltpu.CompilerParams(allow_input_fusion=[True]),
    )(z)
    return out


# speedup vs baseline: 1.1001x; 1.1001x over previous
import functools

import jax
import jax.numpy as jnp
import numpy as np
from jax import lax
from jax.experimental import pallas as pl

_B, _N, _K = 32, 2048, 64
_KS0 = np.uint32(0)
_KS1 = np.uint32(42)
_KS2 = np.uint32(0x1BD11BDA) ^ _KS0 ^ _KS1
_ROT0 = (13, 15, 26, 6)
_ROT1 = (17, 29, 16, 24)


def _rotl(x, r):
    return lax.shift_left(x, np.uint32(r)) | lax.shift_right_logical(
        x, np.uint32(32 - r))


def _threefry2x32(x0, x1):
    x0 = x0 + _KS0
    x1 = x1 + _KS1
    for ks_a, ks_b, bump, rots in (
        (_KS1, _KS2, 1, _ROT0),
        (_KS2, _KS0, 2, _ROT1),
        (_KS0, _KS1, 3, _ROT0),
        (_KS1, _KS2, 4, _ROT1),
        (_KS2, _KS0, 5, _ROT0),
    ):
        for r in rots:
            x0 = x0 + x1
            x1 = _rotl(x1, r)
            x1 = x1 ^ x0
        x0 = x0 + ks_a
        x1 = x1 + ks_b + np.uint32(bump)
    return x0, x1


def _bits_to_gumbel(bits):
    fb = lax.shift_right_logical(bits, np.uint32(9)) | np.uint32(0x3F800000)
    u = lax.bitcast_convert_type(fb, jnp.float32) - 1.0
    return -jnp.log(-jnp.log(u + 1e-20) + 1e-20)


def _onehot(z):
    m = jnp.max(z, axis=-1, keepdims=True)
    iota = lax.broadcasted_iota(jnp.int32, z.shape, z.ndim - 1)
    idx = jnp.min(jnp.where(z == m, iota, z.shape[-1]), axis=-1, keepdims=True)
    return (iota == idx).astype(jnp.float32)


def _gumbel_hard_kernel(bi, x_ref, o_ref):
    i = pl.program_id(0)
    shp = (bi, _N, _K)
    base = (i * (bi * _N * _K)).astype(jnp.uint32)
    flat = (
        lax.broadcasted_iota(jnp.uint32, shp, 0) * np.uint32(_N * _K)
        + lax.broadcasted_iota(jnp.uint32, shp, 1) * np.uint32(_K)
        + lax.broadcasted_iota(jnp.uint32, shp, 2)
        + base
    )
    b0, b1 = _threefry2x32(jnp.zeros(shp, jnp.uint32), flat)
    o_ref[...] = _onehot(x_ref[...] + _bits_to_gumbel(b0 ^ b1))


def kernel(logits):
    bi = 4
    out = pl.pallas_call(
        functools.partial(_gumbel_hard_kernel, bi),
        out_shape=jax.ShapeDtypeStruct((_B, _N, _K), jnp.float32),
        grid=(_B // bi,),
        in_specs=[pl.BlockSpec((bi, _N, _K), lambda i: (i, 0, 0))],
        out_specs=pl.BlockSpec((bi, _N, _K), lambda i: (i, 0, 0)),
    )(logits)
    return out
